# baseline (device time: 1046482 ns/iter reference)
import jax
import jax.numpy as jnp
from jax import lax
from jax.experimental import pallas as pl
from jax.experimental.pallas import tpu as pltpu


def kernel(x):
    m, n = x.shape
    half = n // 2

    def body(x_ref, out_ref, stage_ref, pack_sem, local_sem, send_sem, recv_sem):
        my_x = lax.axis_index("x")
        my_y = lax.axis_index("y")
        my_z = lax.axis_index("z")
        peer_x = 1 - my_x

        barrier_sem = pltpu.get_barrier_semaphore()
        pl.semaphore_signal(
            barrier_sem, inc=1,
            device_id=(peer_x, my_y, my_z),
            device_id_type=pl.DeviceIdType.MESH,
        )
        pl.semaphore_wait(barrier_sem, 1)

        pack = pltpu.make_async_copy(
            x_ref.at[:, pl.ds(peer_x * half, half)],
            stage_ref,
            pack_sem,
        )
        pack.start()

        local = pltpu.make_async_copy(
            x_ref.at[:, pl.ds(my_x * half, half)],
            out_ref.at[pl.ds(my_x * m, m), :],
            local_sem,
        )
        local.start()

        pack.wait()
        rdma = pltpu.make_async_remote_copy(
            src_ref=stage_ref,
            dst_ref=out_ref.at[pl.ds(my_x * m, m), :],
            send_sem=send_sem,
            recv_sem=recv_sem,
            device_id=(peer_x, my_y, my_z),
            device_id_type=pl.DeviceIdType.MESH,
        )
        rdma.start()

        local.wait()
        rdma.wait()

    out, _stage = pl.pallas_call(
        body,
        out_shape=[
            jax.ShapeDtypeStruct((2 * m, half), x.dtype),
            jax.ShapeDtypeStruct((m, half), x.dtype),
        ],
        in_specs=[pl.BlockSpec(memory_space=pl.ANY)],
        out_specs=[
            pl.BlockSpec(memory_space=pl.ANY),
            pl.BlockSpec(memory_space=pl.ANY),
        ],
        scratch_shapes=[
            pltpu.SemaphoreType.DMA,
            pltpu.SemaphoreType.DMA,
            pltpu.SemaphoreType.DMA,
            pltpu.SemaphoreType.DMA,
        ],
        compiler_params=pltpu.CompilerParams(collective_id=0),
    )(x)
    return out


# device time: 536764 ns/iter; 1.9496x vs baseline; 1.9496x over previous
import jax
import jax.numpy as jnp
from jax import lax
from jax.experimental import pallas as pl
from jax.experimental.pallas import tpu as pltpu

N_SPLIT = 8


def kernel(x):
    m, n = x.shape
    half = n // 2
    rows = m // N_SPLIT

    def body(x_ref, out_ref, local_sems, send_sems, recv_sems):
        my_x = lax.axis_index("x")
        my_y = lax.axis_index("y")
        my_z = lax.axis_index("z")
        peer_x = 1 - my_x

        barrier_sem = pltpu.get_barrier_semaphore()
        pl.semaphore_signal(
            barrier_sem, inc=1,
            device_id=(peer_x, my_y, my_z),
            device_id_type=pl.DeviceIdType.MESH,
        )
        pl.semaphore_wait(barrier_sem, 1)

        rdmas = []
        locals_ = []
        for k in range(N_SPLIT):
            r0 = k * rows
            rdma = pltpu.make_async_remote_copy(
                src_ref=x_ref.at[pl.ds(r0, rows), pl.ds(peer_x * half, half)],
                dst_ref=out_ref.at[pl.ds(my_x * m + r0, rows), :],
                send_sem=send_sems.at[k],
                recv_sem=recv_sems.at[k],
                device_id=(peer_x, my_y, my_z),
                device_id_type=pl.DeviceIdType.MESH,
            )
            rdma.start()
            rdmas.append(rdma)

            local = pltpu.make_async_copy(
                x_ref.at[pl.ds(r0, rows), pl.ds(my_x * half, half)],
                out_ref.at[pl.ds(my_x * m + r0, rows), :],
                local_sems.at[k],
            )
            local.start()
            locals_.append(local)

        for local in locals_:
            local.wait()
        for rdma in rdmas:
            rdma.wait()

    return pl.pallas_call(
        body,
        out_shape=jax.ShapeDtypeStruct((2 * m, half), x.dtype),
        in_specs=[pl.BlockSpec(memory_space=pl.ANY)],
        out_specs=pl.BlockSpec(memory_space=pl.ANY),
        scratch_shapes=[
            pltpu.SemaphoreType.DMA((N_SPLIT,)),
            pltpu.SemaphoreType.DMA((N_SPLIT,)),
            pltpu.SemaphoreType.DMA((N_SPLIT,)),
        ],
        compiler_params=pltpu.CompilerParams(collective_id=0),
    )(x)


# device time: 209866 ns/iter; 4.9864x vs baseline; 2.5577x over previous
import jax
import jax.numpy as jnp
from jax import lax
from jax.experimental import pallas as pl
from jax.experimental.pallas import tpu as pltpu

N_CHUNKS = 8


def kernel(x):
    m, n = x.shape
    half = n // 2
    rows = m // N_CHUNKS

    def body(x_ref, out_ref, in_buf, send_buf, keep_buf,
             load_sems, keep_sems, send_sems, recv_sems):
        my_x = lax.axis_index("x")
        my_y = lax.axis_index("y")
        my_z = lax.axis_index("z")
        peer_x = 1 - my_x

        barrier_sem = pltpu.get_barrier_semaphore()
        pl.semaphore_signal(
            barrier_sem, inc=1,
            device_id=(peer_x, my_y, my_z),
            device_id_type=pl.DeviceIdType.MESH,
        )
        pl.semaphore_wait(barrier_sem, 1)

        def load(k):
            return pltpu.make_async_copy(
                x_ref.at[pl.ds(k * rows, rows), :],
                in_buf.at[k % 2],
                load_sems.at[k % 2],
            )

        rdmas = []
        keeps = []
        load(0).start()
        for k in range(N_CHUNKS):
            if k + 1 < N_CHUNKS:
                load(k + 1).start()
            load(k).wait()

            slot = k % 2

            @pl.when(my_x == 0)
            def _():
                send_buf[k] = in_buf[slot, :, half:]
                keep_buf[k] = in_buf[slot, :, :half]

            @pl.when(my_x == 1)
            def _():
                send_buf[k] = in_buf[slot, :, :half]
                keep_buf[k] = in_buf[slot, :, half:]

            rdma = pltpu.make_async_remote_copy(
                src_ref=send_buf.at[k],
                dst_ref=out_ref.at[pl.ds(my_x * m + k * rows, rows), :],
                send_sem=send_sems.at[k],
                recv_sem=recv_sems.at[k],
                device_id=(peer_x, my_y, my_z),
                device_id_type=pl.DeviceIdType.MESH,
            )
            rdma.start()
            rdmas.append(rdma)

            keep = pltpu.make_async_copy(
                keep_buf.at[k],
                out_ref.at[pl.ds(my_x * m + k * rows, rows), :],
                keep_sems.at[k],
            )
            keep.start()
            keeps.append(keep)

        for keep in keeps:
            keep.wait()
        for rdma in rdmas:
            rdma.wait()

    return pl.pallas_call(
        body,
        out_shape=jax.ShapeDtypeStruct((2 * m, half), x.dtype),
        in_specs=[pl.BlockSpec(memory_space=pl.ANY)],
        out_specs=pl.BlockSpec(memory_space=pl.ANY),
        scratch_shapes=[
            pltpu.VMEM((2, rows, n), x.dtype),
            pltpu.VMEM((N_CHUNKS, rows, half), x.dtype),
            pltpu.VMEM((N_CHUNKS, rows, half), x.dtype),
            pltpu.SemaphoreType.DMA((2,)),
            pltpu.SemaphoreType.DMA((N_CHUNKS,)),
            pltpu.SemaphoreType.DMA((N_CHUNKS,)),
            pltpu.SemaphoreType.DMA((N_CHUNKS,)),
        ],
        compiler_params=pltpu.CompilerParams(
            collective_id=0,
            vmem_limit_bytes=60 * 1024 * 1024,
        ),
    )(x)


# device time: 209261 ns/iter; 5.0008x vs baseline; 1.0029x over previous
import jax
import jax.numpy as jnp
from jax import lax
from jax.experimental import pallas as pl
from jax.experimental.pallas import tpu as pltpu

N_CHUNKS = 16


def kernel(x):
    m, n = x.shape
    half = n // 2
    rows = m // N_CHUNKS

    def body(x_ref, out_ref, in_buf, send_buf, keep_buf,
             load_sems, keep_sems, send_sems, recv_sems):
        my_x = lax.axis_index("x")
        my_y = lax.axis_index("y")
        my_z = lax.axis_index("z")
        peer_x = 1 - my_x

        barrier_sem = pltpu.get_barrier_semaphore()
        pl.semaphore_signal(
            barrier_sem, inc=1,
            device_id=(peer_x, my_y, my_z),
            device_id_type=pl.DeviceIdType.MESH,
        )
        pl.semaphore_wait(barrier_sem, 1)

        def load(k):
            return pltpu.make_async_copy(
                x_ref.at[pl.ds(k * rows, rows), :],
                in_buf.at[k % 2],
                load_sems.at[k % 2],
            )

        rdmas = []
        keeps = []
        load(0).start()
        for k in range(N_CHUNKS):
            if k + 1 < N_CHUNKS:
                load(k + 1).start()
            load(k).wait()

            slot = k % 2

            @pl.when(my_x == 0)
            def _():
                send_buf[k] = in_buf[slot, :, half:]
                keep_buf[k] = in_buf[slot, :, :half]

            @pl.when(my_x == 1)
            def _():
                send_buf[k] = in_buf[slot, :, :half]
                keep_buf[k] = in_buf[slot, :, half:]

            rdma = pltpu.make_async_remote_copy(
                src_ref=send_buf.at[k],
                dst_ref=out_ref.at[pl.ds(my_x * m + k * rows, rows), :],
                send_sem=send_sems.at[k],
                recv_sem=recv_sems.at[k],
                device_id=(peer_x, my_y, my_z),
                device_id_type=pl.DeviceIdType.MESH,
            )
            rdma.start()
            rdmas.append(rdma)

            keep = pltpu.make_async_copy(
                keep_buf.at[k],
                out_ref.at[pl.ds(my_x * m + k * rows, rows), :],
                keep_sems.at[k],
            )
            keep.start()
            keeps.append(keep)

        for keep in keeps:
            keep.wait()
        for rdma in rdmas:
            rdma.wait()

    return pl.pallas_call(
        body,
        out_shape=jax.ShapeDtypeStruct((2 * m, half), x.dtype),
        in_specs=[pl.BlockSpec(memory_space=pl.ANY)],
        out_specs=pl.BlockSpec(memory_space=pl.ANY),
        scratch_shapes=[
            pltpu.VMEM((2, rows, n), x.dtype),
            pltpu.VMEM((N_CHUNKS, rows, half), x.dtype),
            pltpu.VMEM((N_CHUNKS, rows, half), x.dtype),
            pltpu.SemaphoreType.DMA((2,)),
            pltpu.SemaphoreType.DMA((N_CHUNKS,)),
            pltpu.SemaphoreType.DMA((N_CHUNKS,)),
            pltpu.SemaphoreType.DMA((N_CHUNKS,)),
        ],
        compiler_params=pltpu.CompilerParams(
            collective_id=0,
            vmem_limit_bytes=60 * 1024 * 1024,
        ),
    )(x)
